# baseline (speedup 1.0000x reference)
import functools

import jax
import jax.numpy as jnp
from jax import lax
from jax.experimental import pallas as pl
from jax.experimental.pallas import tpu as pltpu
from jax.experimental.pallas import tpu_sc as plsc

NUM_NEWS = 1000000
EMBED_DIM = 32
TOTAL = 16384 * 50

NUM_CORES = 2
NUM_SUBCORES = 16
NW = NUM_CORES * NUM_SUBCORES
B_PER_W = TOTAL // NW
CHUNK = 800
NBUF = 4
NCHUNK = B_PER_W // CHUNK
NGROUP = NCHUNK // NBUF

_mesh = plsc.VectorSubcoreMesh(core_axis_name="c", subcore_axis_name="s")


@functools.partial(
    pl.kernel,
    out_type=jax.ShapeDtypeStruct((TOTAL, EMBED_DIM), jnp.float32),
    mesh=_mesh,
    scratch_types=[
        pltpu.VMEM((NCHUNK, CHUNK), jnp.int32),
        pltpu.VMEM((NBUF, CHUNK, EMBED_DIM), jnp.float32),
        [pltpu.SemaphoreType.DMA for _ in range(NBUF)],
        [pltpu.SemaphoreType.DMA for _ in range(NBUF)],
    ],
    compiler_params=pltpu.CompilerParams(use_tc_tiling_on_sc=False),
)
def _gather_kernel(idx_hbm, table_hbm, out_hbm, idx_v, rows_v, gsems, ssems):
    wid = lax.axis_index("s") * NUM_CORES + lax.axis_index("c")
    row0 = wid * NCHUNK

    pltpu.sync_copy(idx_hbm.at[pl.ds(row0, NCHUNK)], idx_v)

    def g_start(c, b):
        pltpu.async_copy(table_hbm.at[idx_v.at[c]], rows_v.at[b], gsems[b])

    def s_start(c, b):
        pltpu.async_copy(rows_v.at[b], out_hbm.at[pl.ds((row0 + c) * CHUNK, CHUNK)], ssems[b])

    def g_wait(b):
        pltpu.make_async_copy(table_hbm.at[idx_v.at[0]], rows_v.at[b], gsems[b]).wait()

    def s_wait(b):
        pltpu.make_async_copy(rows_v.at[b], out_hbm.at[pl.ds(0, CHUNK)], ssems[b]).wait()

    for b in range(NBUF):
        g_start(b, b)

    def body(g, carry):
        for b in range(NBUF):
            g_wait(b)
            s_start(g * NBUF + b, b)
        for b in range(NBUF):
            s_wait(b)
            g_start((g + 1) * NBUF + b, b)
        return carry

    lax.fori_loop(0, NGROUP - 1, body, 0)

    for b in range(NBUF):
        g_wait(b)
        s_start((NGROUP - 1) * NBUF + b, b)
    for b in range(NBUF):
        s_wait(b)


def kernel(news_ids, table):
    idx2d = news_ids.reshape(TOTAL // CHUNK, CHUNK)
    out = _gather_kernel(idx2d, table)
    return out.reshape(news_ids.shape[0], news_ids.shape[1], EMBED_DIM)


# speedup vs baseline: 1.6253x; 1.6253x over previous
import functools

import jax
import jax.numpy as jnp
from jax import lax
from jax.experimental import pallas as pl
from jax.experimental.pallas import tpu as pltpu
from jax.experimental.pallas import tpu_sc as plsc

NUM_NEWS = 1000000
EMBED_DIM = 32
BATCH = 16384
HIST = 50
TOTAL = BATCH * HIST

NW = 32
B_PER_W = TOTAL // NW
CHUNK = 512
NCH = B_PER_W // CHUNK
NTILECOL_PER_W = B_PER_W // 128

_mesh = plsc.VectorSubcoreMesh(core_axis_name="c", subcore_axis_name="s")


@functools.partial(
    pl.kernel,
    out_type=jax.ShapeDtypeStruct((TOTAL // 32, 8, 128), jnp.float32),
    mesh=_mesh,
    scratch_types=[
        pltpu.VMEM((B_PER_W,), jnp.int32),
        pltpu.VMEM((2, CHUNK, EMBED_DIM), jnp.float32),
        pltpu.VMEM((2, EMBED_DIM, CHUNK), jnp.float32),
        [pltpu.SemaphoreType.DMA for _ in range(2)],
        [pltpu.SemaphoreType.DMA for _ in range(2)],
    ],
    compiler_params=pltpu.CompilerParams(
        use_tc_tiling_on_sc=False, needs_layout_passes=False
    ),
)
def _gather_kernel(idx_hbm, table_hbm, out_hbm, idx_v, rows_v, tbuf_v, gsems, ssems):
    wid = lax.axis_index("s") * 2 + lax.axis_index("c")
    base = wid * B_PER_W
    t0 = wid * NTILECOL_PER_W

    pltpu.sync_copy(idx_hbm.at[pl.ds(base, B_PER_W)], idx_v)

    iota = lax.broadcasted_iota(jnp.int32, (16,), 0)

    def g_start(c, b):
        pltpu.async_copy(
            table_hbm.at[idx_v.at[pl.ds(c * CHUNK, CHUNK)]], rows_v.at[b], gsems[b]
        )

    def g_wait(b):
        pltpu.make_async_copy(
            table_hbm.at[idx_v.at[pl.ds(0, CHUNK)]], rows_v.at[b], gsems[b]
        ).wait()

    def transpose(b):
        rows = rows_v.at[b]
        tb = tbuf_v.at[b]

        def body(r4, carry):
            for u in range(4):
                r = r4 * 4 + u
                col = jnp.broadcast_to(r, (16,))
                for dh in range(2):
                    val = rows[r, pl.ds(dh * 16, 16)]
                    plsc.store_scatter(tb, [iota + dh * 16, col], val)
            return carry

        lax.fori_loop(0, CHUNK // 4, body, 0)

    def s_fire(c, b):
        tstart = t0 + c * 4
        l = tstart // 128
        ib0 = tstart - l * 128
        for td in range(4):
            tile0 = (l * 4 + td) * 128 + ib0
            for tc4 in range(4):
                pltpu.async_copy(
                    tbuf_v.at[b, pl.ds(td * 8, 8), pl.ds(tc4 * 128, 128)],
                    out_hbm.at[tile0 + tc4],
                    ssems[b],
                )

    def s_drain(b):
        for _ in range(16):
            pltpu.make_async_copy(
                tbuf_v.at[b, pl.ds(0, 8), pl.ds(0, 128)], out_hbm.at[0], ssems[b]
            ).wait()

    g_start(0, 0)

    def group(g, carry):
        for b in range(2):
            c = g * 2 + b
            g_wait(b)

            @pl.when(c < NCH - 1)
            def _():
                g_start(c + 1, 1 - b)

            @pl.when(g >= 1)
            def _():
                s_drain(b)

            transpose(b)
            s_fire(c, b)
        return carry

    lax.fori_loop(0, NCH // 2, group, 0)
    s_drain(0)
    s_drain(1)


def kernel(news_ids, table):
    idx_flat = jnp.transpose(news_ids).reshape(TOTAL)
    out5 = _gather_kernel(idx_flat, table)
    return (
        out5.reshape(HIST, 4, 128, 8, 128)
        .transpose(2, 4, 0, 1, 3)
        .reshape(BATCH, HIST, EMBED_DIM)
    )


# speedup vs baseline: 2.3434x; 1.4418x over previous
import functools

import jax
import jax.numpy as jnp
from jax import lax
from jax.experimental import pallas as pl
from jax.experimental.pallas import tpu as pltpu
from jax.experimental.pallas import tpu_sc as plsc

NUM_NEWS = 1000000
EMBED_DIM = 32
BATCH = 16384
HIST = 50
TOTAL = BATCH * HIST

NW = 32
B_PER_W = TOTAL // NW
CHUNK = 512
NCH = B_PER_W // CHUNK
NTILECOL_PER_W = B_PER_W // 128

_mesh = plsc.VectorSubcoreMesh(core_axis_name="c", subcore_axis_name="s")


@functools.partial(
    pl.kernel,
    out_type=jax.ShapeDtypeStruct((TOTAL // 32, 8, 128), jnp.float32),
    mesh=_mesh,
    scratch_types=[
        pltpu.VMEM((B_PER_W,), jnp.int32),
        pltpu.VMEM((2, CHUNK, EMBED_DIM), jnp.float32),
        pltpu.VMEM((2, EMBED_DIM, CHUNK + 1), jnp.float32),
        [pltpu.SemaphoreType.DMA for _ in range(2)],
        [pltpu.SemaphoreType.DMA for _ in range(2)],
    ],
    compiler_params=pltpu.CompilerParams(
        use_tc_tiling_on_sc=False, needs_layout_passes=False
    ),
)
def _gather_kernel(idx_hbm, table_hbm, out_hbm, idx_v, rows_v, tbuf_v, gsems, ssems):
    wid = lax.axis_index("s") * 2 + lax.axis_index("c")
    base = wid * B_PER_W
    t0 = wid * NTILECOL_PER_W

    pltpu.sync_copy(idx_hbm.at[pl.ds(base, B_PER_W)], idx_v)

    iota = lax.broadcasted_iota(jnp.int32, (16,), 0)

    def g_start(c, b):
        pltpu.async_copy(
            table_hbm.at[idx_v.at[pl.ds(c * CHUNK, CHUNK)]], rows_v.at[b], gsems[b]
        )

    def g_wait(b):
        pltpu.make_async_copy(
            table_hbm.at[idx_v.at[pl.ds(0, CHUNK)]], rows_v.at[b], gsems[b]
        ).wait()

    def transpose(b):
        rows = rows_v.at[b]
        tb = tbuf_v.at[b]

        def body(r4, carry):
            for u in range(4):
                r = r4 * 4 + u
                col = jnp.broadcast_to(r, (16,))
                for dh in range(2):
                    val = rows[r, pl.ds(dh * 16, 16)]
                    plsc.store_scatter(tb, [iota + dh * 16, col], val)
            return carry

        lax.fori_loop(0, CHUNK // 4, body, 0)

    def s_fire(c, b):
        tstart = t0 + c * 4
        l = tstart // 128
        ib0 = tstart - l * 128
        for td in range(4):
            tile0 = (l * 4 + td) * 128 + ib0
            for tc4 in range(4):
                pltpu.async_copy(
                    tbuf_v.at[b, pl.ds(td * 8, 8), pl.ds(tc4 * 128, 128)],
                    out_hbm.at[tile0 + tc4],
                    ssems[b],
                )

    def s_drain(b):
        for _ in range(16):
            pltpu.make_async_copy(
                tbuf_v.at[b, pl.ds(0, 8), pl.ds(0, 128)], out_hbm.at[0], ssems[b]
            ).wait()

    g_start(0, 0)

    def group(g, carry):
        for b in range(2):
            c = g * 2 + b
            g_wait(b)

            @pl.when(c < NCH - 1)
            def _():
                g_start(c + 1, 1 - b)

            @pl.when(g >= 1)
            def _():
                s_drain(b)

            transpose(b)
            s_fire(c, b)
        return carry

    lax.fori_loop(0, NCH // 2, group, 0)
    s_drain(0)
    s_drain(1)


def kernel(news_ids, table):
    idx_flat = jnp.transpose(news_ids).reshape(TOTAL)
    out5 = _gather_kernel(idx_flat, table)
    return (
        out5.reshape(HIST, 4, 128, 8, 128)
        .transpose(2, 4, 0, 1, 3)
        .reshape(BATCH, HIST, EMBED_DIM)
    )


# speedup vs baseline: 2.3523x; 1.0038x over previous
import functools

import jax
import jax.numpy as jnp
from jax import lax
from jax.experimental import pallas as pl
from jax.experimental.pallas import tpu as pltpu
from jax.experimental.pallas import tpu_sc as plsc

NUM_NEWS = 1000000
EMBED_DIM = 32
BATCH = 16384
HIST = 50
TOTAL = BATCH * HIST

NW = 32
B_PER_W = TOTAL // NW
CHUNK = 512
NCH = B_PER_W // CHUNK
NTILECOL_PER_W = B_PER_W // 128

_mesh = plsc.VectorSubcoreMesh(core_axis_name="c", subcore_axis_name="s")

@functools.partial(
    pl.kernel,
    out_type=jax.ShapeDtypeStruct((TOTAL // 32, 8, 128), jnp.float32),
    mesh=_mesh,
    scratch_types=[
        pltpu.VMEM((B_PER_W,), jnp.int32),
        pltpu.VMEM((2, CHUNK, EMBED_DIM), jnp.float32),
        pltpu.VMEM((2, EMBED_DIM, CHUNK + 1), jnp.float32),
        [pltpu.SemaphoreType.DMA for _ in range(2)],
        [pltpu.SemaphoreType.DMA for _ in range(2)],
    ],
    compiler_params=pltpu.CompilerParams(
        use_tc_tiling_on_sc=False, needs_layout_passes=False
    ),
)
def _gather_kernel(idx_hbm, table_hbm, out_hbm, idx_v, rows_v, tbuf_v, gsems, ssems):
    wid = lax.axis_index("s") * 2 + lax.axis_index("c")
    base = wid * B_PER_W
    t0 = wid * NTILECOL_PER_W

    pltpu.sync_copy(idx_hbm.at[pl.ds(base, B_PER_W)], idx_v)

    iota = lax.broadcasted_iota(jnp.int32, (16,), 0)
    d_lo = iota
    d_hi = iota + 16

    def g_start(c, b):
        pltpu.async_copy(
            table_hbm.at[idx_v.at[pl.ds(c * CHUNK, CHUNK)]], rows_v.at[b], gsems[b]
        )

    def g_wait(b):
        pltpu.make_async_copy(
            table_hbm.at[idx_v.at[pl.ds(0, CHUNK)]], rows_v.at[b], gsems[b]
        ).wait()

    def transpose(b):
        rows = rows_v.at[b]
        tb = tbuf_v.at[b]

        def body(r8, carry):
            for u in range(8):
                r = r8 * 8 + u
                col = jnp.broadcast_to(r, (16,))
                plsc.store_scatter(tb, [d_lo, col], rows[r, pl.ds(0, 16)])
                plsc.store_scatter(tb, [d_hi, col], rows[r, pl.ds(16, 16)])
            return carry

        lax.fori_loop(0, CHUNK // 8, body, 0)

    def s_fire(c, b):
        tstart = t0 + c * 4
        l = tstart // 128
        ib0 = tstart - l * 128
        for td in range(4):
            tile0 = (l * 4 + td) * 128 + ib0
            for tc4 in range(4):
                pltpu.async_copy(
                    tbuf_v.at[b, pl.ds(td * 8, 8), pl.ds(tc4 * 128, 128)],
                    out_hbm.at[tile0 + tc4],
                    ssems[b],
                )

    def s_drain(b):
        for _ in range(16):
            pltpu.make_async_copy(
                tbuf_v.at[b, pl.ds(0, 8), pl.ds(0, 128)], out_hbm.at[0], ssems[b]
            ).wait()

    g_start(0, 0)

    def group(g, carry):
        for b in range(2):
            c = g * 2 + b
            g_wait(b)

            @pl.when(c < NCH - 1)
            def _():
                g_start(c + 1, 1 - b)

            @pl.when(g >= 1)
            def _():
                s_drain(b)

            transpose(b)
            s_fire(c, b)
        return carry

    lax.fori_loop(0, NCH // 2, group, 0)
    s_drain(0)
    s_drain(1)


def kernel(news_ids, table):
    idx_flat = jnp.transpose(news_ids).reshape(TOTAL)
    out5 = _gather_kernel(idx_flat, table)
    return (
        out5.reshape(HIST, 4, 128, 8, 128)
        .transpose(2, 4, 0, 1, 3)
        .reshape(BATCH, HIST, EMBED_DIM)
    )
